# baseline (speedup 1.0000x reference)
import functools

import jax
import jax.numpy as jnp
from jax import lax
from jax.experimental import pallas as pl
from jax.experimental.pallas import tpu as pltpu
from jax.experimental.pallas import tpu_sc as plsc

_N = 10000
_E = 320000
_D = 128

_NC = 2
_NS = 16
_NW = _NC * _NS
_EPW = _E // _NW
_K = 128
_NCH = 78
_KT = 16
_TOFF = _NCH * _K
_NP = 10240
_RPT = _NP // _NS
_ZR = 64
_PAD = 256


def _sc_counts(dst, zrows, ones):
    mesh = plsc.VectorSubcoreMesh(core_axis_name="c", subcore_axis_name="s")

    @functools.partial(
        pl.kernel, mesh=mesh,
        out_type=jax.ShapeDtypeStruct((_NC, _NP, _D), jnp.float32),
        scratch_types=dict(
            acc=pltpu.VMEM_SHARED((_NP, _D), jnp.float32),
            zbuf=pltpu.VMEM((_ZR, _D), jnp.float32),
            onesb=pltpu.VMEM((_K, _D), jnp.float32),
            dstbuf0=pltpu.VMEM((_K,), jnp.int32),
            dstbuf1=pltpu.VMEM((_K,), jnp.int32),
            dstT=pltpu.VMEM((_KT,), jnp.int32),
            semi0=pltpu.SemaphoreType.DMA,
            semi1=pltpu.SemaphoreType.DMA,
            sems0=pltpu.SemaphoreType.DMA,
            sems1=pltpu.SemaphoreType.DMA,
        ),
    )
    def k(dst_hbm, zrows_hbm, ones_hbm, out_hbm,
          acc, zbuf, onesb, dstbuf0, dstbuf1, dstT,
          semi0, semi1, sems0, sems1):
        c = lax.axis_index("c")
        s = lax.axis_index("s")
        wid = s * _NC + c
        tile_base = s * _RPT
        ebase = wid * _EPW

        pltpu.sync_copy(zrows_hbm, zbuf)
        pltpu.sync_copy(ones_hbm, onesb)
        for i in range(_RPT // _ZR):
            pltpu.sync_copy(zbuf, acc.at[pl.ds(tile_base + i * _ZR, _ZR)])
        plsc.subcore_barrier()

        bufs = (dstbuf0, dstbuf1)
        isems = (semi0, semi1)
        ssems = (sems0, sems1)

        def idx_start(chunk, b):
            pltpu.async_copy(dst_hbm.at[pl.ds(ebase + chunk * _K, _K)],
                             bufs[b], isems[b])

        def idx_drain(b):
            pltpu.make_async_copy(dst_hbm.at[pl.ds(ebase, _K)],
                                  bufs[b], isems[b]).wait()

        def sc_start(b):
            pltpu.async_copy(onesb, acc.at[bufs[b]], ssems[b], add=True)

        def sc_drain(b):
            pltpu.make_async_copy(onesb, acc.at[bufs[b]], ssems[b]).wait()

        pltpu.sync_copy(dst_hbm.at[pl.ds(ebase, _K)], dstbuf0)
        idx_start(1, 1)

        @pl.loop(0, _NCH // 2)
        def edge_pair(t):
            cc = 2 * t
            idx_drain(1)
            sc_start(0)
            sc_start(1)
            sc_drain(0)
            idx_start(cc + 2, 0)
            sc_drain(1)
            idx_start(cc + 3, 1)
            idx_drain(0)

        idx_drain(1)
        pltpu.sync_copy(dst_hbm.at[pl.ds(ebase + _TOFF, _KT)], dstT)
        pltpu.sync_copy(onesb.at[pl.ds(0, _KT)], acc.at[dstT], add=True)

        plsc.subcore_barrier()
        for i in range(_RPT // _ZR):
            r0 = tile_base + i * _ZR
            pltpu.sync_copy(acc.at[pl.ds(r0, _ZR)], zbuf)
            pltpu.sync_copy(zbuf, out_hbm.at[c, pl.ds(r0, _ZR)])

    return k(dst, zrows, ones)


_KS = 64
_NQ = 39
_ZRS = 64


def _sc_seg_sum(h, src, dst, zrows):
    mesh = plsc.VectorSubcoreMesh(core_axis_name="c", subcore_axis_name="s")

    scratch = [
        pltpu.VMEM_SHARED((_NP, _D), jnp.float32),
        pltpu.VMEM((_ZRS, _D), jnp.float32),
        pltpu.VMEM((_KT,), jnp.int32),
        pltpu.VMEM((_KT,), jnp.int32),
        pltpu.VMEM((_KT, _D), jnp.float32),
    ]
    for q in range(4):
        scratch += [
            pltpu.VMEM((_KS,), jnp.int32),
            pltpu.VMEM((_KS,), jnp.int32),
            pltpu.VMEM((_KS, _D), jnp.float32),
            pltpu.SemaphoreType.DMA,
            pltpu.SemaphoreType.DMA,
            pltpu.SemaphoreType.DMA,
        ]

    @functools.partial(
        pl.kernel, mesh=mesh,
        out_type=jax.ShapeDtypeStruct((_NC, _NP, _D), jnp.float32),
        scratch_types=scratch,
    )
    def k(h_hbm, src_hbm, dst_hbm, zrows_hbm, out_hbm,
          acc, zbuf, srcT, dstT, rowsT, *qrefs):
        c = lax.axis_index("c")
        s = lax.axis_index("s")
        wid = s * _NC + c
        tile_base = s * _RPT
        ebase = wid * _EPW

        sb = qrefs[0::6]
        db = qrefs[1::6]
        rb = qrefs[2::6]
        ise = qrefs[3::6]
        gse = qrefs[4::6]
        sse = qrefs[5::6]

        pltpu.sync_copy(zrows_hbm, zbuf)
        for i in range(_RPT // _ZRS):
            pltpu.sync_copy(zbuf, acc.at[pl.ds(tile_base + i * _ZRS, _ZRS)])
        plsc.subcore_barrier()

        def idx_start(chunk, q):
            base = ebase + chunk * _KS
            pltpu.async_copy(src_hbm.at[pl.ds(base, _KS)], sb[q], ise[q])
            pltpu.async_copy(dst_hbm.at[pl.ds(base, _KS)], db[q], ise[q])

        def idx_drain(q):
            pltpu.make_async_copy(src_hbm.at[pl.ds(ebase, _KS)],
                                  sb[q], ise[q]).wait()
            pltpu.make_async_copy(dst_hbm.at[pl.ds(ebase, _KS)],
                                  db[q], ise[q]).wait()

        def gstart(q):
            pltpu.async_copy(h_hbm.at[sb[q]], rb[q], gse[q])

        def gdrain(q):
            pltpu.make_async_copy(h_hbm.at[sb[q]], rb[q], gse[q]).wait()

        def sstart(q):
            pltpu.async_copy(rb[q], acc.at[db[q]], sse[q], add=True)

        def sdrain(q):
            pltpu.make_async_copy(rb[q], acc.at[db[q]], sse[q]).wait()

        for q in range(4):
            idx_start(q, q)

        @pl.loop(0, _NQ)
        def quad(t):
            c0 = 4 * t
            idx_drain(0); gstart(0)
            idx_drain(1); gstart(1)
            gdrain(0); sstart(0)
            idx_drain(2); gstart(2)
            gdrain(1); sstart(1)
            sdrain(0); idx_start(c0 + 4, 0)
            idx_drain(3); gstart(3)
            gdrain(2); sstart(2)
            sdrain(1); idx_start(c0 + 5, 1)
            gdrain(3); sstart(3)
            sdrain(2); idx_start(c0 + 6, 2)
            sdrain(3); idx_start(c0 + 7, 3)

        for q in range(4):
            idx_drain(q)
        pltpu.sync_copy(src_hbm.at[pl.ds(ebase + _TOFF, _KT)], srcT)
        pltpu.sync_copy(dst_hbm.at[pl.ds(ebase + _TOFF, _KT)], dstT)
        pltpu.async_copy(h_hbm.at[srcT], rowsT, gse[0]).wait()
        pltpu.sync_copy(rowsT, acc.at[dstT], add=True)

        plsc.subcore_barrier()
        for i in range(_RPT // _ZRS):
            r0 = tile_base + i * _ZRS
            pltpu.sync_copy(acc.at[pl.ds(r0, _ZRS)], zbuf)
            pltpu.sync_copy(zbuf, out_hbm.at[c, pl.ds(r0, _ZRS)])

    return k(h, src, dst, zrows)


def _embed_tc(x, w):
    def body(x_ref, w_ref, o_ref):
        o_ref[...] = jnp.dot(x_ref[...], w_ref[...],
                             preferred_element_type=jnp.float32)

    return pl.pallas_call(
        body, out_shape=jax.ShapeDtypeStruct((_N, _D), jnp.float32)
    )(x, w)


def _dense_tc(h, p0, p1, c0, c1, wt, wb, b, g, be):
    def body(h_ref, p0_ref, p1_ref, c0_ref, c1_ref, wt_ref, wb_ref,
             b_ref, g_ref, be_ref, o_ref):
        hh = h_ref[...]
        cnt = c0_ref[: _N, 0:1] + c1_ref[: _N, 0:1] + 1.0
        aggr = (p0_ref[: _N, :] + p1_ref[: _N, :] + hh) / cnt
        out = (jnp.dot(hh, wt_ref[...], preferred_element_type=jnp.float32)
               + jnp.dot(aggr, wb_ref[...], preferred_element_type=jnp.float32)
               + b_ref[...])
        nrm = jnp.sqrt(jnp.sum(out * out, axis=1, keepdims=True))
        out = out / jnp.maximum(nrm, 1e-12)
        mu = jnp.mean(out, axis=0, keepdims=True)
        var = jnp.mean((out - mu) ** 2, axis=0, keepdims=True)
        out = (out - mu) * (g_ref[...] * lax.rsqrt(var + 1e-5)) + be_ref[...] + hh
        o_ref[...] = jnp.maximum(out, 0.0)

    return pl.pallas_call(
        body, out_shape=jax.ShapeDtypeStruct((_N, _D), jnp.float32)
    )(h, p0, p1, c0, c1, wt, wb, b, g, be)


def kernel(x, edge_index, edge_w, W_emb, p1a, p1b, W1, b1, g1, be1,
           p2a, p2b, W2, b2, g2, be2):
    del edge_w, p1a, p1b, p2a, p2b
    pad = jnp.zeros((2, _PAD), jnp.int32)
    eip = jnp.concatenate([edge_index, pad], axis=1)
    src = eip[0]
    dst = eip[1]
    b1r = b1.reshape(1, _D)
    g1r = g1.reshape(1, _D)
    be1r = be1.reshape(1, _D)
    b2r = b2.reshape(1, _D)
    g2r = g2.reshape(1, _D)
    be2r = be2.reshape(1, _D)
    zrows = jnp.zeros((_ZR, _D), jnp.float32)
    ones = jnp.ones((_K, _D), jnp.float32)

    h = _embed_tc(x, W_emb)
    cnt = _sc_counts(dst, zrows, ones)
    part = _sc_seg_sum(h, src, dst, zrows)
    h1 = _dense_tc(h, part[0], part[1], cnt[0], cnt[1],
                   W1[:_D], W1[_D:], b1r, g1r, be1r)
    part2 = _sc_seg_sum(h1, src, dst, zrows)
    h2 = _dense_tc(h1, part2[0], part2[1], cnt[0], cnt[1],
                   W2[:_D], W2[_D:], b2r, g2r, be2r)
    return h2


# speedup vs baseline: 1.0924x; 1.0924x over previous
import functools

import jax
import jax.numpy as jnp
from jax import lax
from jax.experimental import pallas as pl
from jax.experimental.pallas import tpu as pltpu
from jax.experimental.pallas import tpu_sc as plsc

_N = 10000
_E = 320000
_D = 128

_NC = 2
_NS = 16
_NW = _NC * _NS
_EPW = _E // _NW
_K = 128
_NCH = 78
_KT = 16
_TOFF = _NCH * _K
_NP = 10240
_RPT = _NP // _NS
_ZR = 64
_PAD = 256


def _sc_counts(dst, zrows, ones):
    mesh = plsc.VectorSubcoreMesh(core_axis_name="c", subcore_axis_name="s")

    @functools.partial(
        pl.kernel, mesh=mesh,
        out_type=jax.ShapeDtypeStruct((_NC, _NP, _D), jnp.float32),
        scratch_types=dict(
            acc=pltpu.VMEM_SHARED((_NP, _D), jnp.float32),
            zbuf=pltpu.VMEM((_ZR, _D), jnp.float32),
            onesb=pltpu.VMEM((_K, _D), jnp.float32),
            dstbuf0=pltpu.VMEM((_K,), jnp.int32),
            dstbuf1=pltpu.VMEM((_K,), jnp.int32),
            dstT=pltpu.VMEM((_KT,), jnp.int32),
            semi0=pltpu.SemaphoreType.DMA,
            semi1=pltpu.SemaphoreType.DMA,
            sems0=pltpu.SemaphoreType.DMA,
            sems1=pltpu.SemaphoreType.DMA,
        ),
    )
    def k(dst_hbm, zrows_hbm, ones_hbm, out_hbm,
          acc, zbuf, onesb, dstbuf0, dstbuf1, dstT,
          semi0, semi1, sems0, sems1):
        c = lax.axis_index("c")
        s = lax.axis_index("s")
        wid = s * _NC + c
        tile_base = s * _RPT
        ebase = wid * _EPW

        pltpu.sync_copy(zrows_hbm, zbuf)
        pltpu.sync_copy(ones_hbm, onesb)
        for i in range(_RPT // _ZR):
            pltpu.sync_copy(zbuf, acc.at[pl.ds(tile_base + i * _ZR, _ZR)])
        plsc.subcore_barrier()

        bufs = (dstbuf0, dstbuf1)
        isems = (semi0, semi1)
        ssems = (sems0, sems1)

        def idx_start(chunk, b):
            pltpu.async_copy(dst_hbm.at[pl.ds(ebase + chunk * _K, _K)],
                             bufs[b], isems[b])

        def idx_drain(b):
            pltpu.make_async_copy(dst_hbm.at[pl.ds(ebase, _K)],
                                  bufs[b], isems[b]).wait()

        def sc_start(b):
            pltpu.async_copy(onesb, acc.at[bufs[b]], ssems[b], add=True)

        def sc_drain(b):
            pltpu.make_async_copy(onesb, acc.at[bufs[b]], ssems[b]).wait()

        pltpu.sync_copy(dst_hbm.at[pl.ds(ebase, _K)], dstbuf0)
        idx_start(1, 1)

        @pl.loop(0, _NCH // 2)
        def edge_pair(t):
            cc = 2 * t
            idx_drain(1)
            sc_start(0)
            sc_start(1)
            sc_drain(0)
            idx_start(cc + 2, 0)
            sc_drain(1)
            idx_start(cc + 3, 1)
            idx_drain(0)

        idx_drain(1)
        pltpu.sync_copy(dst_hbm.at[pl.ds(ebase + _TOFF, _KT)], dstT)
        pltpu.sync_copy(onesb.at[pl.ds(0, _KT)], acc.at[dstT], add=True)

        plsc.subcore_barrier()
        for i in range(_RPT // _ZR):
            r0 = tile_base + i * _ZR
            pltpu.sync_copy(acc.at[pl.ds(r0, _ZR)], zbuf)
            pltpu.sync_copy(zbuf, out_hbm.at[c, pl.ds(r0, _ZR)])

    return k(dst, zrows, ones)


def _sc_seg_sum(h, src, dst, zrows):
    mesh = plsc.VectorSubcoreMesh(core_axis_name="c", subcore_axis_name="s")

    @functools.partial(
        pl.kernel, mesh=mesh,
        out_type=jax.ShapeDtypeStruct((_NC, _NP, _D), jnp.float32),
        scratch_types=dict(
            acc=pltpu.VMEM_SHARED((_NP, _D), jnp.float32),
            zbuf=pltpu.VMEM((_ZR, _D), jnp.float32),
            srcbuf0=pltpu.VMEM((_K,), jnp.int32),
            srcbuf1=pltpu.VMEM((_K,), jnp.int32),
            dstbuf0=pltpu.VMEM((_K,), jnp.int32),
            dstbuf1=pltpu.VMEM((_K,), jnp.int32),
            rows0=pltpu.VMEM((_K, _D), jnp.float32),
            rows1=pltpu.VMEM((_K, _D), jnp.float32),
            srcT=pltpu.VMEM((_KT,), jnp.int32),
            dstT=pltpu.VMEM((_KT,), jnp.int32),
            rowsT=pltpu.VMEM((_KT, _D), jnp.float32),
            semi0=pltpu.SemaphoreType.DMA,
            semi1=pltpu.SemaphoreType.DMA,
            semg0=pltpu.SemaphoreType.DMA,
            semg1=pltpu.SemaphoreType.DMA,
        ),
    )
    def k(h_hbm, src_hbm, dst_hbm, zrows_hbm, out_hbm,
          acc, zbuf, srcbuf0, srcbuf1, dstbuf0, dstbuf1,
          rows0, rows1, srcT, dstT, rowsT, semi0, semi1, semg0, semg1):
        c = lax.axis_index("c")
        s = lax.axis_index("s")
        wid = s * _NC + c
        tile_base = s * _RPT
        ebase = wid * _EPW

        pltpu.sync_copy(zrows_hbm, zbuf)
        for i in range(_RPT // _ZR):
            pltpu.sync_copy(zbuf, acc.at[pl.ds(tile_base + i * _ZR, _ZR)])
        plsc.subcore_barrier()

        sbufs = (srcbuf0, srcbuf1)
        dbufs = (dstbuf0, dstbuf1)
        rbufs = (rows0, rows1)
        isems = (semi0, semi1)
        gsems = (semg0, semg1)

        def idx_start(chunk, b):
            base = ebase + chunk * _K
            pltpu.async_copy(src_hbm.at[pl.ds(base, _K)], sbufs[b], isems[b])
            pltpu.async_copy(dst_hbm.at[pl.ds(base, _K)], dbufs[b], isems[b])

        def idx_drain(b):
            pltpu.make_async_copy(src_hbm.at[pl.ds(ebase, _K)],
                                  sbufs[b], isems[b]).wait()
            pltpu.make_async_copy(dst_hbm.at[pl.ds(ebase, _K)],
                                  dbufs[b], isems[b]).wait()

        def gather_start(b):
            pltpu.async_copy(h_hbm.at[sbufs[b]], rbufs[b], gsems[b])

        def gather_drain(b):
            pltpu.make_async_copy(h_hbm.at[sbufs[b]], rbufs[b],
                                  gsems[b]).wait()

        pltpu.sync_copy(src_hbm.at[pl.ds(ebase, _K)], srcbuf0)
        pltpu.sync_copy(dst_hbm.at[pl.ds(ebase, _K)], dstbuf0)
        gather_start(0)
        idx_start(1, 1)

        @pl.loop(0, _NCH // 2)
        def edge_pair(t):
            cc = 2 * t
            idx_drain(1)
            gather_start(1)
            gather_drain(0)
            pltpu.sync_copy(rbufs[0], acc.at[dstbuf0], add=True)
            idx_start(cc + 2, 0)
            idx_drain(0)
            gather_start(0)
            gather_drain(1)
            pltpu.sync_copy(rbufs[1], acc.at[dstbuf1], add=True)
            idx_start(cc + 3, 1)

        gather_drain(0)
        idx_drain(1)
        pltpu.sync_copy(src_hbm.at[pl.ds(ebase + _TOFF, _KT)], srcT)
        pltpu.sync_copy(dst_hbm.at[pl.ds(ebase + _TOFF, _KT)], dstT)
        pltpu.async_copy(h_hbm.at[srcT], rowsT, semg0).wait()
        pltpu.sync_copy(rowsT, acc.at[dstT], add=True)

        plsc.subcore_barrier()
        for i in range(_RPT // _ZR):
            r0 = tile_base + i * _ZR
            pltpu.sync_copy(acc.at[pl.ds(r0, _ZR)], zbuf)
            pltpu.sync_copy(zbuf, out_hbm.at[c, pl.ds(r0, _ZR)])

    return k(h, src, dst, zrows)


_NB = 5
_BR = _N // _NB


def _embed_tc(x, w):
    def body(x_ref, w_ref, o_ref):
        o_ref[...] = jnp.dot(x_ref[...], w_ref[...],
                             preferred_element_type=jnp.float32)

    return pl.pallas_call(
        body,
        grid=(_NB,),
        in_specs=[
            pl.BlockSpec((_BR, _D), lambda i: (i, 0)),
            pl.BlockSpec((_D, _D), lambda i: (0, 0)),
        ],
        out_specs=pl.BlockSpec((_BR, _D), lambda i: (i, 0)),
        out_shape=jax.ShapeDtypeStruct((_N, _D), jnp.float32),
    )(x, w)


def _dense_a(h, p0, p1, c0, c1, wt, wb, b):
    def body(h_ref, p0_ref, p1_ref, c0_ref, c1_ref, wt_ref, wb_ref, b_ref,
             o_ref, st_ref, sacc):
        i = pl.program_id(0)
        hh = h_ref[...]
        cnt = c0_ref[:, 0:1] + c1_ref[:, 0:1] + 1.0
        aggr = (p0_ref[...] + p1_ref[...] + hh) / cnt
        out = (jnp.dot(hh, wt_ref[...], preferred_element_type=jnp.float32)
               + jnp.dot(aggr, wb_ref[...], preferred_element_type=jnp.float32)
               + b_ref[...])
        nrm = jnp.sqrt(jnp.sum(out * out, axis=1, keepdims=True))
        out = out / jnp.maximum(nrm, 1e-12)
        o_ref[...] = out

        @pl.when(i == 0)
        def _():
            sacc[...] = jnp.zeros_like(sacc)

        sacc[0:1, :] += jnp.sum(out, axis=0, keepdims=True)
        sacc[1:2, :] += jnp.sum(out * out, axis=0, keepdims=True)

        @pl.when(i == _NB - 1)
        def _():
            st_ref[...] = sacc[...]

    return pl.pallas_call(
        body,
        grid=(_NB,),
        in_specs=[
            pl.BlockSpec((_BR, _D), lambda i: (i, 0)),
            pl.BlockSpec((_BR, _D), lambda i: (i, 0)),
            pl.BlockSpec((_BR, _D), lambda i: (i, 0)),
            pl.BlockSpec((_BR, _D), lambda i: (i, 0)),
            pl.BlockSpec((_BR, _D), lambda i: (i, 0)),
            pl.BlockSpec((_D, _D), lambda i: (0, 0)),
            pl.BlockSpec((_D, _D), lambda i: (0, 0)),
            pl.BlockSpec((1, _D), lambda i: (0, 0)),
        ],
        out_specs=[
            pl.BlockSpec((_BR, _D), lambda i: (i, 0)),
            pl.BlockSpec((8, _D), lambda i: (0, 0)),
        ],
        out_shape=[
            jax.ShapeDtypeStruct((_N, _D), jnp.float32),
            jax.ShapeDtypeStruct((8, _D), jnp.float32),
        ],
        scratch_shapes=[pltpu.VMEM((8, _D), jnp.float32)],
    )(h, p0, p1, c0, c1, wt, wb, b)


def _dense_b(o, h, stats, g, be):
    def body(o_ref, h_ref, st_ref, g_ref, be_ref, out_ref):
        mu = st_ref[0:1, :] / _N
        ex2 = st_ref[1:2, :] / _N
        var = ex2 - mu * mu
        scale = g_ref[...] * lax.rsqrt(var + 1e-5)
        out_ref[...] = jnp.maximum(
            (o_ref[...] - mu) * scale + be_ref[...] + h_ref[...], 0.0)

    return pl.pallas_call(
        body,
        grid=(_NB,),
        in_specs=[
            pl.BlockSpec((_BR, _D), lambda i: (i, 0)),
            pl.BlockSpec((_BR, _D), lambda i: (i, 0)),
            pl.BlockSpec((8, _D), lambda i: (0, 0)),
            pl.BlockSpec((1, _D), lambda i: (0, 0)),
            pl.BlockSpec((1, _D), lambda i: (0, 0)),
        ],
        out_specs=pl.BlockSpec((_BR, _D), lambda i: (i, 0)),
        out_shape=jax.ShapeDtypeStruct((_N, _D), jnp.float32),
    )(o, h, stats, g, be)


def _dense_tc(h, p0, p1, c0, c1, wt, wb, b, g, be):
    o, stats = _dense_a(h, p0, p1, c0, c1, wt, wb, b)
    return _dense_b(o, h, stats, g, be)


def kernel(x, edge_index, edge_w, W_emb, p1a, p1b, W1, b1, g1, be1,
           p2a, p2b, W2, b2, g2, be2):
    del edge_w, p1a, p1b, p2a, p2b
    pad = jnp.zeros((2, _PAD), jnp.int32)
    eip = jnp.concatenate([edge_index, pad], axis=1)
    src = eip[0]
    dst = eip[1]
    b1r = b1.reshape(1, _D)
    g1r = g1.reshape(1, _D)
    be1r = be1.reshape(1, _D)
    b2r = b2.reshape(1, _D)
    g2r = g2.reshape(1, _D)
    be2r = be2.reshape(1, _D)
    zrows = jnp.zeros((_ZR, _D), jnp.float32)
    ones = jnp.ones((_K, _D), jnp.float32)

    h = _embed_tc(x, W_emb)
    cnt = _sc_counts(dst, zrows, ones)
    part = _sc_seg_sum(h, src, dst, zrows)
    h1 = _dense_tc(h, part[0], part[1], cnt[0], cnt[1],
                   W1[:_D], W1[_D:], b1r, g1r, be1r)
    part2 = _sc_seg_sum(h1, src, dst, zrows)
    h2 = _dense_tc(h1, part2[0], part2[1], cnt[0], cnt[1],
                   W2[:_D], W2[_D:], b2r, g2r, be2r)
    return h2


# speedup vs baseline: 1.1537x; 1.0561x over previous
import functools

import jax
import jax.numpy as jnp
from jax import lax
from jax.experimental import pallas as pl
from jax.experimental.pallas import tpu as pltpu
from jax.experimental.pallas import tpu_sc as plsc

_N = 10000
_E = 320000
_D = 128

_NC = 2
_NS = 16
_NW = _NC * _NS
_EPW = _E // _NW
_K = 128
_NCH = 78
_KT = 16
_TOFF = _NCH * _K
_NP = 10240
_RPT = _NP // _NS
_ZR = 64
_PAD = 256


def _sc_counts(dst, zrows, ones):
    mesh = plsc.VectorSubcoreMesh(core_axis_name="c", subcore_axis_name="s")

    @functools.partial(
        pl.kernel, mesh=mesh,
        out_type=jax.ShapeDtypeStruct((_NC, _NP, _D), jnp.float32),
        scratch_types=dict(
            acc=pltpu.VMEM_SHARED((_NP, _D), jnp.float32),
            zbuf=pltpu.VMEM((_ZR, _D), jnp.float32),
            onesb=pltpu.VMEM((_K, _D), jnp.float32),
            dstbuf0=pltpu.VMEM((_K,), jnp.int32),
            dstbuf1=pltpu.VMEM((_K,), jnp.int32),
            dstbuf2=pltpu.VMEM((_K,), jnp.int32),
            dstbuf3=pltpu.VMEM((_K,), jnp.int32),
            dstT=pltpu.VMEM((_KT,), jnp.int32),
            semi0=pltpu.SemaphoreType.DMA,
            semi1=pltpu.SemaphoreType.DMA,
            semi2=pltpu.SemaphoreType.DMA,
            semi3=pltpu.SemaphoreType.DMA,
            sems0=pltpu.SemaphoreType.DMA,
            sems1=pltpu.SemaphoreType.DMA,
            sems2=pltpu.SemaphoreType.DMA,
            sems3=pltpu.SemaphoreType.DMA,
        ),
    )
    def k(dst_hbm, zrows_hbm, ones_hbm, out_hbm,
          acc, zbuf, onesb, dstbuf0, dstbuf1, dstbuf2, dstbuf3, dstT,
          semi0, semi1, semi2, semi3, sems0, sems1, sems2, sems3):
        c = lax.axis_index("c")
        s = lax.axis_index("s")
        wid = s * _NC + c
        tile_base = s * _RPT
        ebase = wid * _EPW

        pltpu.sync_copy(zrows_hbm, zbuf)
        pltpu.sync_copy(ones_hbm, onesb)
        for i in range(_RPT // _ZR):
            pltpu.sync_copy(zbuf, acc.at[pl.ds(tile_base + i * _ZR, _ZR)])
        plsc.subcore_barrier()

        bufs = (dstbuf0, dstbuf1, dstbuf2, dstbuf3)
        isems = (semi0, semi1, semi2, semi3)
        ssems = (sems0, sems1, sems2, sems3)

        def idx_start(chunk, b):
            pltpu.async_copy(dst_hbm.at[pl.ds(ebase + chunk * _K, _K)],
                             bufs[b], isems[b])

        def idx_drain(b):
            pltpu.make_async_copy(dst_hbm.at[pl.ds(ebase, _K)],
                                  bufs[b], isems[b]).wait()

        def sc_start(b):
            pltpu.async_copy(onesb, acc.at[bufs[b]], ssems[b], add=True)

        def sc_drain(b):
            pltpu.make_async_copy(onesb, acc.at[bufs[b]], ssems[b]).wait()

        for q in range(4):
            idx_start(q, q)

        @pl.loop(0, 19)
        def edge_quad(t):
            c0 = 4 * t
            idx_drain(0); sc_start(0)
            idx_drain(1); sc_start(1)
            idx_drain(2); sc_start(2)
            sc_drain(0); idx_start(c0 + 4, 0)
            idx_drain(3); sc_start(3)
            sc_drain(1); idx_start(c0 + 5, 1)
            sc_drain(2); idx_start(c0 + 6, 2)
            sc_drain(3); idx_start(c0 + 7, 3)

        idx_drain(0); sc_start(0)
        idx_drain(1); sc_start(1)
        sc_drain(0)
        sc_drain(1)
        idx_drain(2)
        idx_drain(3)
        pltpu.sync_copy(dst_hbm.at[pl.ds(ebase + _TOFF, _KT)], dstT)
        pltpu.sync_copy(onesb.at[pl.ds(0, _KT)], acc.at[dstT], add=True)

        plsc.subcore_barrier()
        for i in range(_RPT // _ZR):
            r0 = tile_base + i * _ZR
            pltpu.sync_copy(acc.at[pl.ds(r0, _ZR)], zbuf)
            pltpu.sync_copy(zbuf, out_hbm.at[c, pl.ds(r0, _ZR)])

    return k(dst, zrows, ones)


def _sc_seg_sum(h, src, dst, zrows):
    mesh = plsc.VectorSubcoreMesh(core_axis_name="c", subcore_axis_name="s")

    @functools.partial(
        pl.kernel, mesh=mesh,
        out_type=jax.ShapeDtypeStruct((_NC, _NP, _D), jnp.float32),
        scratch_types=dict(
            acc=pltpu.VMEM_SHARED((_NP, _D), jnp.float32),
            zbuf=pltpu.VMEM((_ZR, _D), jnp.float32),
            srcbuf0=pltpu.VMEM((_K,), jnp.int32),
            srcbuf1=pltpu.VMEM((_K,), jnp.int32),
            dstbuf0=pltpu.VMEM((_K,), jnp.int32),
            dstbuf1=pltpu.VMEM((_K,), jnp.int32),
            rows0=pltpu.VMEM((_K, _D), jnp.float32),
            rows1=pltpu.VMEM((_K, _D), jnp.float32),
            srcT=pltpu.VMEM((_KT,), jnp.int32),
            dstT=pltpu.VMEM((_KT,), jnp.int32),
            rowsT=pltpu.VMEM((_KT, _D), jnp.float32),
            semi0=pltpu.SemaphoreType.DMA,
            semi1=pltpu.SemaphoreType.DMA,
            semg0=pltpu.SemaphoreType.DMA,
            semg1=pltpu.SemaphoreType.DMA,
        ),
    )
    def k(h_hbm, src_hbm, dst_hbm, zrows_hbm, out_hbm,
          acc, zbuf, srcbuf0, srcbuf1, dstbuf0, dstbuf1,
          rows0, rows1, srcT, dstT, rowsT, semi0, semi1, semg0, semg1):
        c = lax.axis_index("c")
        s = lax.axis_index("s")
        wid = s * _NC + c
        tile_base = s * _RPT
        ebase = wid * _EPW

        pltpu.sync_copy(zrows_hbm, zbuf)
        for i in range(_RPT // _ZR):
            pltpu.sync_copy(zbuf, acc.at[pl.ds(tile_base + i * _ZR, _ZR)])
        plsc.subcore_barrier()

        sbufs = (srcbuf0, srcbuf1)
        dbufs = (dstbuf0, dstbuf1)
        rbufs = (rows0, rows1)
        isems = (semi0, semi1)
        gsems = (semg0, semg1)

        def idx_start(chunk, b):
            base = ebase + chunk * _K
            pltpu.async_copy(src_hbm.at[pl.ds(base, _K)], sbufs[b], isems[b])
            pltpu.async_copy(dst_hbm.at[pl.ds(base, _K)], dbufs[b], isems[b])

        def idx_drain(b):
            pltpu.make_async_copy(src_hbm.at[pl.ds(ebase, _K)],
                                  sbufs[b], isems[b]).wait()
            pltpu.make_async_copy(dst_hbm.at[pl.ds(ebase, _K)],
                                  dbufs[b], isems[b]).wait()

        def gather_start(b):
            pltpu.async_copy(h_hbm.at[sbufs[b]], rbufs[b], gsems[b])

        def gather_drain(b):
            pltpu.make_async_copy(h_hbm.at[sbufs[b]], rbufs[b],
                                  gsems[b]).wait()

        pltpu.sync_copy(src_hbm.at[pl.ds(ebase, _K)], srcbuf0)
        pltpu.sync_copy(dst_hbm.at[pl.ds(ebase, _K)], dstbuf0)
        gather_start(0)
        idx_start(1, 1)

        @pl.loop(0, _NCH // 2)
        def edge_pair(t):
            cc = 2 * t
            idx_drain(1)
            gather_start(1)
            gather_drain(0)
            pltpu.sync_copy(rbufs[0], acc.at[dstbuf0], add=True)
            idx_start(cc + 2, 0)
            idx_drain(0)
            gather_start(0)
            gather_drain(1)
            pltpu.sync_copy(rbufs[1], acc.at[dstbuf1], add=True)
            idx_start(cc + 3, 1)

        gather_drain(0)
        idx_drain(1)
        pltpu.sync_copy(src_hbm.at[pl.ds(ebase + _TOFF, _KT)], srcT)
        pltpu.sync_copy(dst_hbm.at[pl.ds(ebase + _TOFF, _KT)], dstT)
        pltpu.async_copy(h_hbm.at[srcT], rowsT, semg0).wait()
        pltpu.sync_copy(rowsT, acc.at[dstT], add=True)

        plsc.subcore_barrier()
        for i in range(_RPT // _ZR):
            r0 = tile_base + i * _ZR
            pltpu.sync_copy(acc.at[pl.ds(r0, _ZR)], zbuf)
            pltpu.sync_copy(zbuf, out_hbm.at[c, pl.ds(r0, _ZR)])

    return k(h, src, dst, zrows)


def _embed_tc(x, w):
    def body(x_ref, w_ref, o_ref):
        o_ref[...] = jnp.dot(x_ref[...], w_ref[...],
                             preferred_element_type=jnp.float32)

    return pl.pallas_call(
        body, out_shape=jax.ShapeDtypeStruct((_N, _D), jnp.float32)
    )(x, w)


def _dense_tc(h, p0, p1, c0, c1, wt, wb, b, g, be):
    def body(h_ref, p0_ref, p1_ref, c0_ref, c1_ref, wt_ref, wb_ref,
             b_ref, g_ref, be_ref, o_ref):
        hh = h_ref[...]
        cnt = c0_ref[: _N, 0:1] + c1_ref[: _N, 0:1] + 1.0
        aggr = (p0_ref[: _N, :] + p1_ref[: _N, :] + hh) / cnt
        out = (jnp.dot(hh, wt_ref[...], preferred_element_type=jnp.float32)
               + jnp.dot(aggr, wb_ref[...], preferred_element_type=jnp.float32)
               + b_ref[...])
        nrm = jnp.sqrt(jnp.sum(out * out, axis=1, keepdims=True))
        out = out / jnp.maximum(nrm, 1e-12)
        mu = jnp.mean(out, axis=0, keepdims=True)
        var = jnp.mean((out - mu) ** 2, axis=0, keepdims=True)
        out = (out - mu) * (g_ref[...] * lax.rsqrt(var + 1e-5)) + be_ref[...] + hh
        o_ref[...] = jnp.maximum(out, 0.0)

    return pl.pallas_call(
        body, out_shape=jax.ShapeDtypeStruct((_N, _D), jnp.float32)
    )(h, p0, p1, c0, c1, wt, wb, b, g, be)


def kernel(x, edge_index, edge_w, W_emb, p1a, p1b, W1, b1, g1, be1,
           p2a, p2b, W2, b2, g2, be2):
    del edge_w, p1a, p1b, p2a, p2b
    pad = jnp.zeros((2, _PAD), jnp.int32)
    eip = jnp.concatenate([edge_index, pad], axis=1)
    src = eip[0]
    dst = eip[1]
    b1r = b1.reshape(1, _D)
    g1r = g1.reshape(1, _D)
    be1r = be1.reshape(1, _D)
    b2r = b2.reshape(1, _D)
    g2r = g2.reshape(1, _D)
    be2r = be2.reshape(1, _D)
    zrows = jnp.zeros((_ZR, _D), jnp.float32)
    ones = jnp.ones((_K, _D), jnp.float32)

    h = _embed_tc(x, W_emb)
    cnt = _sc_counts(dst, zrows, ones)
    part = _sc_seg_sum(h, src, dst, zrows)
    h1 = _dense_tc(h, part[0], part[1], cnt[0], cnt[1],
                   W1[:_D], W1[_D:], b1r, g1r, be1r)
    part2 = _sc_seg_sum(h1, src, dst, zrows)
    h2 = _dense_tc(h1, part2[0], part2[1], cnt[0], cnt[1],
                   W2[:_D], W2[_D:], b2r, g2r, be2r)
    return h2
